# Initial kernel scaffold; baseline (speedup 1.0000x reference)
import jax
import jax.numpy as jnp
from jax.experimental import pallas as pl


def kernel(paper_x, author_x, paper_batch, author_batch, W_p, b_p, W_a, b_a, W_h, b_h):
    raise NotImplementedError("write your pallas kernel here")


# speedup vs baseline: 2.3957x; 2.3957x over previous
import functools

import jax
import jax.numpy as jnp
from jax import lax
from jax.experimental import pallas as pl
from jax.experimental.pallas import tpu as pltpu
from jax.experimental.pallas import tpu_sc as plsc

N = 1_600_000
G = 4096
NW = 32
RW = N // NW
C = 10_000
NCHUNK = RW // C
GROUPS = C // 16
ACC = 5 * G


def _sc_body(px, pb, ax, ab, out, xbuf, idbuf, acc):
    wid = lax.axis_index("s") * 2 + lax.axis_index("c")
    iota = lax.iota(jnp.int32, 16)
    iota4 = iota * 4
    ones = jnp.ones((16,), jnp.float32)
    zeros = jnp.zeros((16,), jnp.float32)

    for t, (xh, bh) in enumerate(((px, pb), (ax, ab))):
        def zero_body(i, carry):
            acc[pl.ds(i * 16, 16)] = zeros
            return carry
        lax.fori_loop(0, ACC // 16, zero_body, 0)

        for c in range(NCHUNK):
            r0 = wid * RW + c * C
            pltpu.sync_copy(xh.at[pl.ds(r0 * 4, C * 4)], xbuf)
            pltpu.sync_copy(bh.at[pl.ds(r0, C)], idbuf)

            def grp(g, carry):
                rg = g * 16
                ids = idbuf[pl.ds(rg, 16)]
                for d in range(4):
                    xf = plsc.load_gather(xbuf, [rg * 4 + d + iota4])
                    plsc.addupdate_scatter(acc, [ids + d * G], xf)
                plsc.addupdate_scatter(acc, [ids + 4 * G], ones)
                return carry
            lax.fori_loop(0, GROUPS, grp, 0)

        pltpu.sync_copy(acc, out.at[pl.ds((wid * 2 + t) * ACC, ACC)])


def _sc_partials(px, pb, ax, ab):
    mesh = plsc.VectorSubcoreMesh(core_axis_name="c", subcore_axis_name="s")
    return pl.kernel(
        _sc_body,
        mesh=mesh,
        out_type=jax.ShapeDtypeStruct((NW * 2 * ACC,), jnp.float32),
        scratch_types=[
            pltpu.VMEM((C * 4,), jnp.float32),
            pltpu.VMEM((C,), jnp.int32),
            pltpu.VMEM((ACC,), jnp.float32),
        ],
        compiler_params=pltpu.CompilerParams(needs_layout_passes=False),
    )(px, pb, ax, ab)


def _finish_body(p_ref, ap_ref, aa_ref, wh_ref, bh_ref, o_ref):
    tot = jnp.sum(p_ref[...], axis=0)
    ap = ap_ref[...]
    aa = aa_ref[...]
    wh = wh_ref[...]
    h = []
    for t, a in ((0, ap), (1, aa)):
        s = tot[t * 5:(t + 1) * 5]
        hsum = a[:, 0:1] * s[0:1]
        for d in range(1, 5):
            hsum = hsum + a[:, d:d + 1] * s[d:d + 1]
        h.append(hsum / jnp.maximum(s[4:5], 1.0))
    hcat = jnp.concatenate(h, axis=0)
    o = bh_ref[...] + wh[:, 0:1] * hcat[0:1]
    for j in range(1, 8):
        o = o + wh[:, j:j + 1] * hcat[j:j + 1]
    o_ref[...] = o


def _finish(p, ap, aa, whT, bh2):
    return pl.pallas_call(
        _finish_body,
        out_shape=jax.ShapeDtypeStruct((2, G), jnp.float32),
    )(p, ap, aa, whT, bh2)


@jax.jit
def kernel(paper_x, author_x, paper_batch, author_batch,
           W_p, b_p, W_a, b_a, W_h, b_h):
    px = paper_x.reshape(-1)
    ax = author_x.reshape(-1)
    pb = paper_batch.astype(jnp.int32)
    ab = author_batch.astype(jnp.int32)
    partials = _sc_partials(px, pb, ax, ab)
    p = partials.reshape(NW, 10, G)
    ap = jnp.concatenate([W_p.T, b_p[:, None]], axis=1)
    aa = jnp.concatenate([W_a.T, b_a[:, None]], axis=1)
    out2 = _finish(p, ap, aa, W_h.T, b_h[:, None])
    return out2.T
